# baseline (device time: 107007 ns/iter reference)
import jax
import jax.numpy as jnp
from jax import lax
from jax.experimental import pallas as pl
from jax.experimental.pallas import tpu as pltpu

N_CHUNKS = 8

COMPUTE_MODE = "einsum_b"


def kernel(Q, K, V):
    b, sq, h, d = Q.shape
    skv = K.shape[1]
    kc = skv // N_CHUNKS
    scale = d ** -0.5

    def body(q_ref, k_ref, v_ref, o_ref,
             acc_o, acc_l, send_buf, recv_buf, send_sem, recv_sem):
        step = pl.program_id(0)

        q = q_ref[:, 0, :, :]
        k_blk = k_ref[...]
        v_blk = v_ref[...]

        if COMPUTE_MODE == "einsum_full":
            s = jnp.einsum("bhd,bkhd->bhk", q, k_blk,
                           preferred_element_type=jnp.float32) * scale
            p = jnp.exp(s)
            l_c = jnp.sum(p, axis=2)
            o_c = jnp.einsum("bhk,bkhd->bhd", p, v_blk,
                             preferred_element_type=jnp.float32)
        elif COMPUTE_MODE == "einsum_b":
            l_rows = []
            o_rows = []
            for bi in range(b):
                s_b = lax.dot_general(
                    q[bi], k_blk[bi],
                    dimension_numbers=(((1,), (2,)), ((0,), (1,))),
                    preferred_element_type=jnp.float32,
                ) * scale
                p_b = jnp.exp(s_b)
                l_rows.append(jnp.sum(p_b, axis=1))
                o_rows.append(lax.dot_general(
                    p_b, v_blk[bi],
                    dimension_numbers=(((1,), (0,)), ((0,), (1,))),
                    preferred_element_type=jnp.float32,
                ))
            l_c = jnp.stack(l_rows)
            o_c = jnp.stack(o_rows)
        else:
            s = jnp.sum(q[:, None, :, :] * k_blk, axis=-1) * scale
            p = jnp.exp(s)
            l_c = jnp.sum(p, axis=1)
            o_c = jnp.sum(p[..., None] * v_blk, axis=1)

        @pl.when(step == 0)
        def _():
            acc_o[...] = o_c
            acc_l[...] = l_c

        @pl.when(step != 0)
        def _():
            acc_o[...] += o_c
            acc_l[...] += l_c

        @pl.when(step == N_CHUNKS - 1)
        def _():
            my_x = lax.axis_index("x")
            my_y = lax.axis_index("y")
            my_z = lax.axis_index("z")
            partner = (my_x, 1 - my_y, my_z)

            barrier_sem = pltpu.get_barrier_semaphore()
            pl.semaphore_signal(
                barrier_sem, inc=1, device_id=partner,
                device_id_type=pl.DeviceIdType.MESH,
            )
            pl.semaphore_wait(barrier_sem, 1)

            send_buf[0, :, :, :] = acc_o[...]
            send_buf[1, :, :, :] = jnp.broadcast_to(
                acc_l[...][:, :, None], (b, h, d))

            rdma = pltpu.make_async_remote_copy(
                src_ref=send_buf,
                dst_ref=recv_buf,
                send_sem=send_sem,
                recv_sem=recv_sem,
                device_id=partner,
                device_id_type=pl.DeviceIdType.MESH,
            )
            rdma.start()
            rdma.wait()

            o_tot = send_buf[0, :, :, :] + recv_buf[0, :, :, :]
            l_tot = send_buf[1, :, :, :] + recv_buf[1, :, :, :]
            o_ref[:, 0, :, :] = o_tot / l_tot

    return pl.pallas_call(
        body,
        grid=(N_CHUNKS,),
        out_shape=jax.ShapeDtypeStruct((b, sq, h, d), jnp.float32),
        in_specs=[
            pl.BlockSpec((b, sq, h, d), lambda i: (0, 0, 0, 0)),
            pl.BlockSpec((b, kc, h, d), lambda i: (0, i, 0, 0)),
            pl.BlockSpec((b, kc, h, d), lambda i: (0, i, 0, 0)),
        ],
        out_specs=pl.BlockSpec((b, sq, h, d), lambda i: (0, 0, 0, 0)),
        scratch_shapes=[
            pltpu.VMEM((b, h, d), jnp.float32),
            pltpu.VMEM((b, h), jnp.float32),
            pltpu.VMEM((2, b, h, d), jnp.float32),
            pltpu.VMEM((2, b, h, d), jnp.float32),
            pltpu.SemaphoreType.DMA,
            pltpu.SemaphoreType.DMA,
        ],
        compiler_params=pltpu.CompilerParams(
            collective_id=0,
            dimension_semantics=("arbitrary",),
        ),
    )(Q, K, V)


# device time: 34973 ns/iter; 3.0597x vs baseline; 3.0597x over previous
import jax
import jax.numpy as jnp
from jax import lax
from jax.experimental import pallas as pl
from jax.experimental.pallas import tpu as pltpu

N_CHUNKS = 8

COMPUTE_MODE = "elementwise"


def kernel(Q, K, V):
    b, sq, h, d = Q.shape
    skv = K.shape[1]
    kc = skv // N_CHUNKS
    scale = d ** -0.5

    def body(q_ref, k_ref, v_ref, o_ref,
             acc_o, acc_l, send_buf, recv_buf, send_sem, recv_sem):
        step = pl.program_id(0)

        q = q_ref[:, 0, :, :]
        k_blk = k_ref[...]
        v_blk = v_ref[...]

        if COMPUTE_MODE == "einsum_full":
            s = jnp.einsum("bhd,bkhd->bhk", q, k_blk,
                           preferred_element_type=jnp.float32) * scale
            p = jnp.exp(s)
            l_c = jnp.sum(p, axis=2)
            o_c = jnp.einsum("bhk,bkhd->bhd", p, v_blk,
                             preferred_element_type=jnp.float32)
        elif COMPUTE_MODE == "einsum_b":
            l_rows = []
            o_rows = []
            for bi in range(b):
                s_b = lax.dot_general(
                    q[bi], k_blk[bi],
                    dimension_numbers=(((1,), (2,)), ((0,), (1,))),
                    preferred_element_type=jnp.float32,
                ) * scale
                p_b = jnp.exp(s_b)
                l_rows.append(jnp.sum(p_b, axis=1))
                o_rows.append(lax.dot_general(
                    p_b, v_blk[bi],
                    dimension_numbers=(((1,), (0,)), ((0,), (1,))),
                    preferred_element_type=jnp.float32,
                ))
            l_c = jnp.stack(l_rows)
            o_c = jnp.stack(o_rows)
        else:
            s = jnp.sum(q[:, None, :, :] * k_blk, axis=-1) * scale
            p = jnp.exp(s)
            l_c = jnp.sum(p, axis=1)
            o_c = jnp.sum(p[..., None] * v_blk, axis=1)

        @pl.when(step == 0)
        def _():
            acc_o[...] = o_c
            acc_l[...] = l_c

        @pl.when(step != 0)
        def _():
            acc_o[...] += o_c
            acc_l[...] += l_c

        @pl.when(step == N_CHUNKS - 1)
        def _():
            my_x = lax.axis_index("x")
            my_y = lax.axis_index("y")
            my_z = lax.axis_index("z")
            partner = (my_x, 1 - my_y, my_z)

            barrier_sem = pltpu.get_barrier_semaphore()
            pl.semaphore_signal(
                barrier_sem, inc=1, device_id=partner,
                device_id_type=pl.DeviceIdType.MESH,
            )
            pl.semaphore_wait(barrier_sem, 1)

            send_buf[0, :, :, :] = acc_o[...]
            send_buf[1, :, :, :] = jnp.broadcast_to(
                acc_l[...][:, :, None], (b, h, d))

            rdma = pltpu.make_async_remote_copy(
                src_ref=send_buf,
                dst_ref=recv_buf,
                send_sem=send_sem,
                recv_sem=recv_sem,
                device_id=partner,
                device_id_type=pl.DeviceIdType.MESH,
            )
            rdma.start()
            rdma.wait()

            o_tot = send_buf[0, :, :, :] + recv_buf[0, :, :, :]
            l_tot = send_buf[1, :, :, :] + recv_buf[1, :, :, :]
            o_ref[:, 0, :, :] = o_tot / l_tot

    return pl.pallas_call(
        body,
        grid=(N_CHUNKS,),
        out_shape=jax.ShapeDtypeStruct((b, sq, h, d), jnp.float32),
        in_specs=[
            pl.BlockSpec((b, sq, h, d), lambda i: (0, 0, 0, 0)),
            pl.BlockSpec((b, kc, h, d), lambda i: (0, i, 0, 0)),
            pl.BlockSpec((b, kc, h, d), lambda i: (0, i, 0, 0)),
        ],
        out_specs=pl.BlockSpec((b, sq, h, d), lambda i: (0, 0, 0, 0)),
        scratch_shapes=[
            pltpu.VMEM((b, h, d), jnp.float32),
            pltpu.VMEM((b, h), jnp.float32),
            pltpu.VMEM((2, b, h, d), jnp.float32),
            pltpu.VMEM((2, b, h, d), jnp.float32),
            pltpu.SemaphoreType.DMA,
            pltpu.SemaphoreType.DMA,
        ],
        compiler_params=pltpu.CompilerParams(
            collective_id=0,
            dimension_semantics=("arbitrary",),
        ),
    )(Q, K, V)
